# baseline (speedup 1.0000x reference)
import functools

import jax
import jax.numpy as jnp
from jax.experimental import pallas as pl
from jax.experimental.pallas import tpu as pltpu

_NC = 80
_B = 3
_LBL_C = _NC + 5
_PRD_C = _NC + 5 * _B


def _iou_rows(lx, ly, lw, lh, px, py, pw, ph):
    ax1, ax2 = lx - lw * 0.5, lx + lw * 0.5
    ay1, ay2 = ly - lh * 0.5, ly + lh * 0.5
    bx1, bx2 = px - pw * 0.5, px + pw * 0.5
    by1, by2 = py - ph * 0.5, py + ph * 0.5
    iw = jnp.maximum(jnp.minimum(ax2, bx2) - jnp.maximum(ax1, bx1), 0.0)
    ih = jnp.maximum(jnp.minimum(ay2, by2) - jnp.maximum(ay1, by1), 0.0)
    inter = iw * ih
    union = lw * lh + pw * ph - inter + 1e-6
    return inter / union


def _sqrt_scale(x):
    return jnp.sign(x) * jnp.sqrt(jnp.abs(x))


def _body(lbl_ref, prd_ref, out_ref):
    i = pl.program_id(0)
    n = lbl_ref.shape[0] * 28 * 28
    lbl = lbl_ref[...].reshape(n, _LBL_C)
    prd = prd_ref[...].reshape(n, _PRD_C)

    lq = jnp.transpose(lbl[:, _NC:_NC + 5])
    pq = jnp.transpose(prd[:, _NC:_NC + 5 * _B])

    conf = lq[0:1, :]
    lx, ly, lw, lh = lq[1:2, :], lq[2:3, :], lq[3:4, :], lq[4:5, :]
    pc = [pq[5 * j + 0:5 * j + 1, :] for j in range(_B)]
    px = [pq[5 * j + 1:5 * j + 2, :] for j in range(_B)]
    py = [pq[5 * j + 2:5 * j + 3, :] for j in range(_B)]
    pw = [pq[5 * j + 3:5 * j + 4, :] for j in range(_B)]
    ph = [pq[5 * j + 4:5 * j + 5, :] for j in range(_B)]

    mask_obj = (conf > 0.5).astype(jnp.float32)
    mask_no = (conf != 1.0).astype(jnp.float32)

    ious = [_iou_rows(lx, ly, lw, lh, px[j], py[j], pw[j], ph[j])
            for j in range(_B)]

    best_i, bc, bx, by, bw, bh = ious[0], pc[0], px[0], py[0], pw[0], ph[0]
    for j in range(1, _B):
        keep = best_i >= ious[j]
        best_i = jnp.where(keep, best_i, ious[j])
        bc = jnp.where(keep, bc, pc[j])
        bx = jnp.where(keep, bx, px[j])
        by = jnp.where(keep, by, py[j])
        bw = jnp.where(keep, bw, pw[j])
        bh = jnp.where(keep, bh, ph[j])

    loc = jnp.sum(mask_obj * ((lx - bx) ** 2 + (ly - by) ** 2))
    size = jnp.sum(mask_obj * ((_sqrt_scale(lw) - _sqrt_scale(bw)) ** 2
                               + (_sqrt_scale(lh) - _sqrt_scale(bh)) ** 2))
    pobj = jnp.sum(mask_obj * (conf - bc) ** 2)
    pno = jnp.sum(mask_no * ((conf - pc[0]) ** 2 + (conf - pc[1]) ** 2
                             + (conf - pc[2]) ** 2))

    mask_obj_col = (lbl[:, _NC:_NC + 1] > 0.5).astype(jnp.float32)
    d = lbl - prd[:, :_LBL_C]
    per_lane = jax.lax.dot_general(
        d * d, mask_obj_col, (((0,), (0,)), ((), ())),
        preferred_element_type=jnp.float32)
    lane = jax.lax.broadcasted_iota(jnp.int32, (_LBL_C, 1), 0)
    cls = jnp.sum(jnp.where(lane < _NC, per_lane, 0.0))

    m = 256 * 28 * 28
    s_mb = 1.0 / (m + _B)
    s_mc = 1.0 / (m + _NC)
    lane2 = jax.lax.broadcasted_iota(jnp.int32, (1, 8, 128), 2)
    v = ((lane2 == 0) * (loc * s_mb) + (lane2 == 1) * (size * s_mb)
         + (lane2 == 2) * (pobj * s_mb) + (lane2 == 3) * (pno * s_mb)
         + (lane2 == 4) * (cls * s_mc))
    out_ref[...] = v.astype(jnp.float32)


@functools.partial(jax.jit, static_argnames=("interpret",))
def _run(label, pred, interpret=False):
    bb = 8
    grid = label.shape[0] // bb
    out = pl.pallas_call(
        _body,
        grid=(grid,),
        in_specs=[
            pl.BlockSpec((bb, 28, 28, _LBL_C), lambda i: (i, 0, 0, 0)),
            pl.BlockSpec((bb, 28, 28, _PRD_C), lambda i: (i, 0, 0, 0)),
        ],
        out_specs=pl.BlockSpec((1, 8, 128), lambda i: (i, 0, 0)),
        out_shape=jax.ShapeDtypeStruct((grid, 8, 128), jnp.float32),
        interpret=interpret,
    )(label, pred)
    tot = jnp.sum(out[:, 0, :], axis=0)
    return (tot[0], tot[1], tot[2], tot[3], tot[4])


def kernel(label, pred):
    return _run(label, pred)


# speedup vs baseline: 1.0291x; 1.0291x over previous
import functools

import jax
import jax.numpy as jnp
from jax.experimental import pallas as pl
from jax.experimental.pallas import tpu as pltpu

_NC = 80
_B = 3
_LBL_C = _NC + 5
_PRD_C = _NC + 5 * _B


def _iou_rows(lx, ly, lw, lh, px, py, pw, ph):
    ax1, ax2 = lx - lw * 0.5, lx + lw * 0.5
    ay1, ay2 = ly - lh * 0.5, ly + lh * 0.5
    bx1, bx2 = px - pw * 0.5, px + pw * 0.5
    by1, by2 = py - ph * 0.5, py + ph * 0.5
    iw = jnp.maximum(jnp.minimum(ax2, bx2) - jnp.maximum(ax1, bx1), 0.0)
    ih = jnp.maximum(jnp.minimum(ay2, by2) - jnp.maximum(ay1, by1), 0.0)
    inter = iw * ih
    union = lw * lh + pw * ph - inter + 1e-6
    return inter / union


def _sqrt_scale(x):
    return jnp.sign(x) * jnp.sqrt(jnp.abs(x))


def _body(lbl_ref, prd_ref, out_ref):
    i = pl.program_id(0)
    n = lbl_ref.shape[0] * 28 * 28
    lbl = lbl_ref[...].reshape(n, _LBL_C)
    prd = prd_ref[...].reshape(n, _PRD_C)

    lq = jnp.transpose(lbl[:, _NC:_NC + 5])
    pq = jnp.transpose(prd[:, _NC:_NC + 5 * _B])

    conf = lq[0:1, :]
    lx, ly, lw, lh = lq[1:2, :], lq[2:3, :], lq[3:4, :], lq[4:5, :]
    pc = [pq[5 * j + 0:5 * j + 1, :] for j in range(_B)]
    px = [pq[5 * j + 1:5 * j + 2, :] for j in range(_B)]
    py = [pq[5 * j + 2:5 * j + 3, :] for j in range(_B)]
    pw = [pq[5 * j + 3:5 * j + 4, :] for j in range(_B)]
    ph = [pq[5 * j + 4:5 * j + 5, :] for j in range(_B)]

    mask_obj = (conf > 0.5).astype(jnp.float32)
    mask_no = (conf != 1.0).astype(jnp.float32)

    ious = [_iou_rows(lx, ly, lw, lh, px[j], py[j], pw[j], ph[j])
            for j in range(_B)]

    best_i, bc, bx, by, bw, bh = ious[0], pc[0], px[0], py[0], pw[0], ph[0]
    for j in range(1, _B):
        keep = best_i >= ious[j]
        best_i = jnp.where(keep, best_i, ious[j])
        bc = jnp.where(keep, bc, pc[j])
        bx = jnp.where(keep, bx, px[j])
        by = jnp.where(keep, by, py[j])
        bw = jnp.where(keep, bw, pw[j])
        bh = jnp.where(keep, bh, ph[j])

    loc = jnp.sum(mask_obj * ((lx - bx) ** 2 + (ly - by) ** 2))
    size = jnp.sum(mask_obj * ((_sqrt_scale(lw) - _sqrt_scale(bw)) ** 2
                               + (_sqrt_scale(lh) - _sqrt_scale(bh)) ** 2))
    pobj = jnp.sum(mask_obj * (conf - bc) ** 2)
    pno = jnp.sum(mask_no * ((conf - pc[0]) ** 2 + (conf - pc[1]) ** 2
                             + (conf - pc[2]) ** 2))

    mask_obj_col = (lbl[:, _NC:_NC + 1] > 0.5).astype(jnp.float32)
    d = lbl - prd[:, :_LBL_C]
    per_lane = jax.lax.dot_general(
        d * d, mask_obj_col, (((0,), (0,)), ((), ())),
        preferred_element_type=jnp.float32)
    lane = jax.lax.broadcasted_iota(jnp.int32, (_LBL_C, 1), 0)
    cls = jnp.sum(jnp.where(lane < _NC, per_lane, 0.0))

    m = 256 * 28 * 28
    s_mb = 1.0 / (m + _B)
    s_mc = 1.0 / (m + _NC)
    lane2 = jax.lax.broadcasted_iota(jnp.int32, (1, 8, 128), 2)
    v = ((lane2 == 0) * (loc * s_mb) + (lane2 == 1) * (size * s_mb)
         + (lane2 == 2) * (pobj * s_mb) + (lane2 == 3) * (pno * s_mb)
         + (lane2 == 4) * (cls * s_mc))
    out_ref[...] = v.astype(jnp.float32)


@functools.partial(jax.jit, static_argnames=("interpret",))
def _run(label, pred, interpret=False):
    bb = 16
    grid = label.shape[0] // bb
    out = pl.pallas_call(
        _body,
        grid=(grid,),
        in_specs=[
            pl.BlockSpec((bb, 28, 28, _LBL_C), lambda i: (i, 0, 0, 0)),
            pl.BlockSpec((bb, 28, 28, _PRD_C), lambda i: (i, 0, 0, 0)),
        ],
        out_specs=pl.BlockSpec((1, 8, 128), lambda i: (i, 0, 0)),
        out_shape=jax.ShapeDtypeStruct((grid, 8, 128), jnp.float32),
        interpret=interpret,
    )(label, pred)
    tot = jnp.sum(out[:, 0, :], axis=0)
    return (tot[0], tot[1], tot[2], tot[3], tot[4])


def kernel(label, pred):
    return _run(label, pred)
